# baseline (speedup 1.0000x reference)
import functools
import math

import jax
import jax.numpy as jnp
from jax import lax
from jax.experimental import pallas as pl
from jax.experimental.pallas import tpu as pltpu
from jax.experimental.pallas import tpu_sc as plsc

D_MODEL = 64
LANES = 16
NUM_CORES = 2
NUM_SUBCORES = 16
NUM_WORKERS = NUM_CORES * NUM_SUBCORES
G = 128
K = 4
CHUNK = G * K
SCALE = math.sqrt(D_MODEL)


def _sc_embedding(x2d, lut):
    num_groups = x2d.shape[0]
    b_total = num_groups * G
    groups_per_w = num_groups // NUM_WORKERS
    n_chunks = groups_per_w // K

    mesh = plsc.VectorSubcoreMesh(core_axis_name="c", subcore_axis_name="s")

    @functools.partial(
        pl.kernel,
        out_type=jax.ShapeDtypeStruct((b_total, D_MODEL), jnp.float32),
        mesh=mesh,
        scratch_types=[
            pltpu.VMEM((K, G), jnp.int32),
            pltpu.VMEM((CHUNK, D_MODEL), jnp.float32),
            pltpu.SemaphoreType.DMA,
        ],
        compiler_params=pltpu.CompilerParams(use_tc_tiling_on_sc=False),
    )
    def body(x_hbm, lut_hbm, out_hbm, idx_v, rows_v, sem):
        wid = lax.axis_index("s") * NUM_CORES + lax.axis_index("c")
        g0 = wid * groups_per_w

        @pl.loop(0, n_chunks)
        def _chunk(s):
            gbase = g0 + s * K
            pltpu.sync_copy(x_hbm.at[pl.ds(gbase, K)], idx_v)
            copies = [
                pltpu.async_copy(
                    lut_hbm.at[idx_v.at[j]],
                    rows_v.at[pl.ds(j * G, G)],
                    sem,
                )
                for j in range(K)
            ]
            for cp in copies:
                cp.wait()

            @pl.loop(0, CHUNK)
            def _scale(i):
                for j in range(D_MODEL // LANES):
                    sl = pl.ds(j * LANES, LANES)
                    rows_v[i, sl] = rows_v[i, sl] * SCALE

            pltpu.sync_copy(rows_v, out_hbm.at[pl.ds(gbase * G, CHUNK)])

    return body(x2d, lut)


def kernel(x, lut):
    b, t = x.shape
    x2d = x.reshape(-1).astype(jnp.int32).reshape(-1, G)
    out = _sc_embedding(x2d, lut)
    return out.reshape(b, t, D_MODEL)


# speedup vs baseline: 1.1321x; 1.1321x over previous
import functools
import math

import jax
import jax.numpy as jnp
from jax import lax
from jax.experimental import pallas as pl
from jax.experimental.pallas import tpu as pltpu
from jax.experimental.pallas import tpu_sc as plsc

D_MODEL = 64
LANES = 16
NUM_CORES = 2
NUM_SUBCORES = 16
NUM_WORKERS = NUM_CORES * NUM_SUBCORES
G = 128
K = 4
CHUNK = G * K
NBUF = 3
SCALE = math.sqrt(D_MODEL)


def _sc_embedding(x2d, lut):
    num_groups = x2d.shape[0]
    b_total = num_groups * G
    groups_per_w = num_groups // NUM_WORKERS
    n_chunks = groups_per_w // K

    mesh = plsc.VectorSubcoreMesh(
        core_axis_name="c", subcore_axis_name="s",
        num_cores=NUM_CORES, num_subcores=NUM_SUBCORES,
    )

    @functools.partial(
        pl.kernel,
        out_type=jax.ShapeDtypeStruct((b_total, D_MODEL), jnp.float32),
        mesh=mesh,
        scratch_types=[
            pltpu.VMEM((groups_per_w, G), jnp.int32),
            pltpu.VMEM((NBUF * CHUNK, D_MODEL), jnp.float32),
            [pltpu.SemaphoreType.DMA] * NBUF,
            [pltpu.SemaphoreType.DMA] * NBUF,
        ],
        compiler_params=pltpu.CompilerParams(use_tc_tiling_on_sc=False),
    )
    def body(x_hbm, lut_hbm, out_hbm, idx_all, rows_v, sem_g, sem_o):
        wid = lax.axis_index("s") * NUM_CORES + lax.axis_index("c")
        g0 = wid * groups_per_w

        def rows_at(b):
            return rows_v.at[pl.ds(b * CHUNK, CHUNK)]

        def fire(cur, b):
            for j in range(K):
                pltpu.async_copy(
                    lut_hbm.at[idx_all.at[cur * K + j]],
                    rows_v.at[pl.ds(b * CHUNK + j * G, G)],
                    sem_g[b],
                )

        def drain_gathers(b):
            for j in range(K):
                pltpu.make_async_copy(
                    out_hbm.at[pl.ds(0, G)],
                    rows_v.at[pl.ds(b * CHUNK + j * G, G)],
                    sem_g[b],
                ).wait()

        def drain_out(b):
            pltpu.make_async_copy(
                out_hbm.at[pl.ds(0, CHUNK)], rows_at(b), sem_o[b]
            ).wait()

        def out_write(cur, b):
            pltpu.async_copy(
                rows_at(b),
                out_hbm.at[pl.ds((g0 + cur * K) * G, CHUNK)],
                sem_o[b],
            )

        def scale(b):
            @plsc.parallel_loop(0, CHUNK, unroll=8)
            def _scale(i):
                for j in range(D_MODEL // LANES):
                    sl = pl.ds(j * LANES, LANES)
                    rows_v[b * CHUNK + i, sl] = rows_v[b * CHUNK + i, sl] * SCALE

        pltpu.sync_copy(x_hbm.at[pl.ds(g0, groups_per_w)], idx_all)

        fire(0, 0)
        fire(1, 1)

        n_outer = (n_chunks + NBUF - 1) // NBUF

        @pl.loop(0, n_outer * NBUF, step=NBUF)
        def _outer(s):
            for b in range(NBUF):
                cur = s + b
                bf = (b + 2) % NBUF

                @pl.when(jnp.logical_and(cur + 2 < n_chunks, cur >= 1))
                def _():
                    drain_out(bf)
                    fire(cur + 2, bf)

                @pl.when(jnp.logical_and(cur + 2 < n_chunks, cur < 1))
                def _():
                    fire(cur + 2, bf)

                @pl.when(cur < n_chunks)
                def _():
                    drain_gathers(b)
                    scale(b)
                    out_write(cur, b)

        for b in range(NBUF):
            drain_out(b)

    return body(x2d, lut)


def kernel(x, lut):
    b, t = x.shape
    x2d = x.reshape(-1).astype(jnp.int32).reshape(-1, G)
    out = _sc_embedding(x2d, lut)
    return out.reshape(b, t, D_MODEL)
